# baseline (speedup 1.0000x reference)
import jax
import jax.numpy as jnp
from jax import lax
from jax.experimental import pallas as pl
from jax.experimental.pallas import tpu as pltpu

_K = 8


_BPB = 8


def _router_kernel(x_ref, fc1W_ref, fc1b_ref, gateWT_ref, gatebT_ref, out_ref):
    for bi in range(_BPB):
        xb = x_ref[bi]
        h = jnp.dot(xb, fc1W_ref[...],
                    preferred_element_type=jnp.float32) + fc1b_ref[...]
        lT = lax.dot_general(gateWT_ref[...], h, (((1,), (1,)), ((), ())),
                             preferred_element_type=jnp.float32)
        lT = lT + gatebT_ref[...]
        m = jnp.max(lT, axis=1, keepdims=True)
        ex = jnp.exp(lT - m)
        gpT = ex / jnp.sum(ex, axis=1, keepdims=True)
        S = gpT.shape[1]
        lane = lax.broadcasted_iota(jnp.int32, gpT.shape, 1)
        work = gpT
        pieces = []
        probs_cols = []
        for _ in range(_K):
            mk = jnp.max(work, axis=1, keepdims=True)
            ik = jnp.min(jnp.where(work == mk, lane, S),
                         axis=1, keepdims=True)
            sel = (lane == ik)
            Mk = jnp.where(sel, mk, 0.0)
            pieces.append(jnp.dot(Mk, h, preferred_element_type=jnp.float32))
            probs_cols.append(mk)
            work = jnp.where(sel, -jnp.inf, work)
        probs_e = jnp.concatenate(probs_cols, axis=1)
        inp_full = jnp.concatenate(pieces + [probs_e], axis=1)
        out_ref[bi] = inp_full


_EPB = 8


def _expert_kernel(inp_ref, eW_ref, eb_ref, fc2W_ref, fc2b_ref,
                   fc3W_ref, fc3b_ref, W4r_ref, fc4b_ref,
                   fc5W_ref, fc5b_ref, fc6W_ref, fc6b_ref,
                   out_ref, acc_ref):
    g = pl.program_id(0)
    contrib = None
    for j in range(_EPB):
        inp = inp_ref[:, j, :]
        probs = inp[:, 256:264]
        p = jnp.sum(probs, axis=1, keepdims=True)
        out = jnp.dot(inp, eW_ref[j],
                      preferred_element_type=jnp.float32) + eb_ref[j]
        out = out * p
        moe = jnp.maximum(jnp.concatenate([out, probs], axis=1), 0.0)
        y = jnp.maximum(jnp.dot(moe, fc2W_ref[...],
                                preferred_element_type=jnp.float32)
                        + fc2b_ref[...], 0.0)
        y = jnp.maximum(jnp.dot(y, fc3W_ref[...],
                                preferred_element_type=jnp.float32)
                        + fc3b_ref[...], 0.0)
        c = jnp.dot(y, W4r_ref[j], preferred_element_type=jnp.float32)
        contrib = c if contrib is None else contrib + c

    @pl.when(g == 0)
    def _():
        acc_ref[...] = contrib

    @pl.when(g != 0)
    def _():
        acc_ref[...] = acc_ref[...] + contrib

    @pl.when(g == pl.num_programs(0) - 1)
    def _():
        z = jnp.maximum(acc_ref[...] + fc4b_ref[...], 0.0)
        z = jnp.maximum(jnp.dot(z, fc5W_ref[...],
                                preferred_element_type=jnp.float32)
                        + fc5b_ref[...], 0.0)
        out_ref[...] = jnp.dot(z, fc6W_ref[...],
                               preferred_element_type=jnp.float32) + fc6b_ref[...]


@jax.jit
def kernel(x, fc1_W, fc1_b, gate_W, gate_b, expert_W, expert_b,
           fc2_W, fc2_b, fc3_W, fc3_b, fc4_W, fc4_b,
           fc5_W, fc5_b, fc6_W, fc6_b):
    B, S, TOK = x.shape
    E = gate_W.shape[1]
    IW = 32 * _K + _K
    O = expert_W.shape[-1]

    inp_full = pl.pallas_call(
        _router_kernel,
        grid=(B // _BPB,),
        in_specs=[
            pl.BlockSpec((_BPB, S, TOK), lambda b: (b, 0, 0)),
            pl.BlockSpec((TOK, 32), lambda b: (0, 0)),
            pl.BlockSpec((1, 32), lambda b: (0, 0)),
            pl.BlockSpec((E, 32), lambda b: (0, 0)),
            pl.BlockSpec((E, 1), lambda b: (0, 0)),
        ],
        out_specs=pl.BlockSpec((_BPB, E, IW), lambda b: (b, 0, 0)),
        out_shape=jax.ShapeDtypeStruct((B, E, IW), jnp.float32),
    )(x, fc1_W, fc1_b.reshape(1, 32), gate_W.T, gate_b.reshape(E, 1))

    W4r = fc4_W.reshape(E, 128, 128)
    out = pl.pallas_call(
        _expert_kernel,
        grid=(E // _EPB,),
        in_specs=[
            pl.BlockSpec((B, _EPB, IW), lambda g: (0, g, 0)),
            pl.BlockSpec((_EPB, IW, O), lambda g: (g, 0, 0)),
            pl.BlockSpec((_EPB, 1, O), lambda g: (g, 0, 0)),
            pl.BlockSpec((32, 128), lambda g: (0, 0)),
            pl.BlockSpec((1, 128), lambda g: (0, 0)),
            pl.BlockSpec((128, 128), lambda g: (0, 0)),
            pl.BlockSpec((1, 128), lambda g: (0, 0)),
            pl.BlockSpec((_EPB, 128, 128), lambda g: (g, 0, 0)),
            pl.BlockSpec((1, 128), lambda g: (0, 0)),
            pl.BlockSpec((128, 128), lambda g: (0, 0)),
            pl.BlockSpec((1, 128), lambda g: (0, 0)),
            pl.BlockSpec((128, 10), lambda g: (0, 0)),
            pl.BlockSpec((1, 10), lambda g: (0, 0)),
        ],
        out_specs=pl.BlockSpec((B, 10), lambda g: (0, 0)),
        out_shape=jax.ShapeDtypeStruct((B, 10), jnp.float32),
        scratch_shapes=[pltpu.VMEM((B, 128), jnp.float32)],
    )(inp_full, expert_W, expert_b.reshape(E, 1, O), fc2_W,
      fc2_b.reshape(1, 128), fc3_W, fc3_b.reshape(1, 128), W4r,
      fc4_b.reshape(1, 128), fc5_W, fc5_b.reshape(1, 128),
      fc6_W, fc6_b.reshape(1, 10))
    return out


# speedup vs baseline: 1.6578x; 1.6578x over previous
import jax
import jax.numpy as jnp
from jax import lax
from jax.experimental import pallas as pl
from jax.experimental.pallas import tpu as pltpu

_K = 8
_BPB = 8


def _router_kernel(x_ref, fc1W_ref, fc1b_ref, gateWT_ref, gatebT_ref, out_ref):
    S = x_ref.shape[1]
    E = gateWT_ref.shape[0]
    xall = x_ref[...].reshape(_BPB * S, x_ref.shape[2])
    hall = jnp.dot(xall, fc1W_ref[...],
                   preferred_element_type=jnp.float32) + fc1b_ref[...]
    pieces = []
    for bi in range(_BPB):
        hb = hall[bi * S:(bi + 1) * S]
        pieces.append(lax.dot_general(
            gateWT_ref[...], hb, (((1,), (1,)), ((), ())),
            preferred_element_type=jnp.float32))
    lT = jnp.concatenate(pieces, axis=0)
    lT = lT + jnp.concatenate([gatebT_ref[...]] * _BPB, axis=0)
    m = jnp.max(lT, axis=1, keepdims=True)
    ex = jnp.exp(lT - m)
    gpT = ex / jnp.sum(ex, axis=1, keepdims=True)
    lane = lax.broadcasted_iota(jnp.int32, gpT.shape, 1)
    work = gpT
    Ms = []
    probs_cols = []
    for _ in range(_K):
        mk = jnp.max(work, axis=1, keepdims=True)
        ik = jnp.min(jnp.where(work == mk, lane, S),
                     axis=1, keepdims=True)
        sel = (lane == ik)
        Ms.append(jnp.where(sel, mk, 0.0))
        probs_cols.append(mk)
        work = jnp.where(sel, -jnp.inf, work)
    probs_all = jnp.concatenate(probs_cols, axis=1)
    for bi in range(_BPB):
        hb = hall[bi * S:(bi + 1) * S]
        cols = [jnp.dot(Mk[bi * E:(bi + 1) * E], hb,
                        preferred_element_type=jnp.float32) for Mk in Ms]
        cols.append(probs_all[bi * E:(bi + 1) * E])
        out_ref[bi] = jnp.concatenate(cols, axis=1)


_EPB = 8


def _expert_kernel(inp_ref, eW_ref, eb_ref, fc2W_ref, fc2b_ref,
                   fc3W_ref, fc3b_ref, W4r_ref, fc4b_ref,
                   fc5W_ref, fc5b_ref, fc6W_ref, fc6b_ref,
                   out_ref, acc_ref):
    g = pl.program_id(0)
    contrib = None
    for j in range(_EPB):
        inp = inp_ref[:, j, :]
        probs = inp[:, 256:264]
        p = jnp.sum(probs, axis=1, keepdims=True)
        out = jnp.dot(inp, eW_ref[j],
                      preferred_element_type=jnp.float32) + eb_ref[j]
        out = out * p
        moe = jnp.maximum(jnp.concatenate([out, probs], axis=1), 0.0)
        y = jnp.maximum(jnp.dot(moe, fc2W_ref[...],
                                preferred_element_type=jnp.float32)
                        + fc2b_ref[...], 0.0)
        y = jnp.maximum(jnp.dot(y, fc3W_ref[...],
                                preferred_element_type=jnp.float32)
                        + fc3b_ref[...], 0.0)
        c = jnp.dot(y, W4r_ref[j], preferred_element_type=jnp.float32)
        contrib = c if contrib is None else contrib + c

    @pl.when(g == 0)
    def _():
        acc_ref[...] = contrib

    @pl.when(g != 0)
    def _():
        acc_ref[...] = acc_ref[...] + contrib

    @pl.when(g == pl.num_programs(0) - 1)
    def _():
        z = jnp.maximum(acc_ref[...] + fc4b_ref[...], 0.0)
        z = jnp.maximum(jnp.dot(z, fc5W_ref[...],
                                preferred_element_type=jnp.float32)
                        + fc5b_ref[...], 0.0)
        out_ref[...] = jnp.dot(z, fc6W_ref[...],
                               preferred_element_type=jnp.float32) + fc6b_ref[...]


@jax.jit
def kernel(x, fc1_W, fc1_b, gate_W, gate_b, expert_W, expert_b,
           fc2_W, fc2_b, fc3_W, fc3_b, fc4_W, fc4_b,
           fc5_W, fc5_b, fc6_W, fc6_b):
    B, S, TOK = x.shape
    E = gate_W.shape[1]
    IW = 32 * _K + _K
    O = expert_W.shape[-1]

    inp_full = pl.pallas_call(
        _router_kernel,
        grid=(B // _BPB,),
        in_specs=[
            pl.BlockSpec((_BPB, S, TOK), lambda b: (b, 0, 0)),
            pl.BlockSpec((TOK, 32), lambda b: (0, 0)),
            pl.BlockSpec((1, 32), lambda b: (0, 0)),
            pl.BlockSpec((E, 32), lambda b: (0, 0)),
            pl.BlockSpec((E, 1), lambda b: (0, 0)),
        ],
        out_specs=pl.BlockSpec((_BPB, E, IW), lambda b: (b, 0, 0)),
        out_shape=jax.ShapeDtypeStruct((B, E, IW), jnp.float32),
    )(x, fc1_W, fc1_b.reshape(1, 32), gate_W.T, gate_b.reshape(E, 1))

    W4r = fc4_W.reshape(E, 128, 128)
    out = pl.pallas_call(
        _expert_kernel,
        grid=(E // _EPB,),
        in_specs=[
            pl.BlockSpec((B, _EPB, IW), lambda g: (0, g, 0)),
            pl.BlockSpec((_EPB, IW, O), lambda g: (g, 0, 0)),
            pl.BlockSpec((_EPB, 1, O), lambda g: (g, 0, 0)),
            pl.BlockSpec((32, 128), lambda g: (0, 0)),
            pl.BlockSpec((1, 128), lambda g: (0, 0)),
            pl.BlockSpec((128, 128), lambda g: (0, 0)),
            pl.BlockSpec((1, 128), lambda g: (0, 0)),
            pl.BlockSpec((_EPB, 128, 128), lambda g: (g, 0, 0)),
            pl.BlockSpec((1, 128), lambda g: (0, 0)),
            pl.BlockSpec((128, 128), lambda g: (0, 0)),
            pl.BlockSpec((1, 128), lambda g: (0, 0)),
            pl.BlockSpec((128, 10), lambda g: (0, 0)),
            pl.BlockSpec((1, 10), lambda g: (0, 0)),
        ],
        out_specs=pl.BlockSpec((B, 10), lambda g: (0, 0)),
        out_shape=jax.ShapeDtypeStruct((B, 10), jnp.float32),
        scratch_shapes=[pltpu.VMEM((B, 128), jnp.float32)],
    )(inp_full, expert_W, expert_b.reshape(E, 1, O), fc2_W,
      fc2_b.reshape(1, 128), fc3_W, fc3_b.reshape(1, 128), W4r,
      fc4_b.reshape(1, 128), fc5_W, fc5_b.reshape(1, 128),
      fc6_W, fc6_b.reshape(1, 10))
    return out
